# baseline (speedup 1.0000x reference)
import functools

import jax
import jax.numpy as jnp
from jax import lax
from jax.experimental import pallas as pl
from jax.experimental.pallas import tpu as pltpu
from jax.experimental.pallas import tpu_sc as plsc

_NUM_CORES = 2
_NUM_SUBCORES = 16
_NW = _NUM_CORES * _NUM_SUBCORES
_LANES = 16
_NBUF = 4


def _make_sc_pool(B, L, EMB, group):
    assert B % (_NW * 2 * group) == 0 and EMB % _LANES == 0
    assert group * L <= 128
    bpw = B // _NW
    gpw = bpw // group
    gl = group * L
    inv_l = 1.0 / float(L)
    mesh = plsc.VectorSubcoreMesh(core_axis_name="c", subcore_axis_name="s")

    @functools.partial(
        pl.kernel,
        out_type=jax.ShapeDtypeStruct((B, EMB), jnp.float32),
        mesh=mesh,
        scratch_types=[
            pltpu.VMEM((gpw, gl), jnp.int32),
            pltpu.VMEM((bpw, EMB), jnp.float32),
            pltpu.VMEM((gl, EMB), jnp.float32),
            pltpu.VMEM((gl, EMB), jnp.float32),
            pltpu.SemaphoreType.DMA,
            pltpu.SemaphoreType.DMA,
        ],
    )
    def sc_pool(text_hbm, table_hbm, out_hbm, idx_v, out_v, buf0, buf1, sem0, sem1):
        wid = lax.axis_index("c") * _NUM_SUBCORES + lax.axis_index("s")
        pltpu.sync_copy(text_hbm.at[pl.ds(wid * gpw, gpw)], idx_v)

        def accumulate(buf, g):
            for e in range(group):
                accs = [
                    buf[e * L, pl.ds(cb * _LANES, _LANES)]
                    for cb in range(EMB // _LANES)
                ]
                for r in range(1, L):
                    for cb in range(EMB // _LANES):
                        accs[cb] = accs[cb] + buf[e * L + r, pl.ds(cb * _LANES, _LANES)]
                for cb in range(EMB // _LANES):
                    out_v[g * group + e, pl.ds(cb * _LANES, _LANES)] = accs[cb] * inv_l

        pltpu.async_copy(table_hbm.at[idx_v.at[0]], buf0, sem0)

        @pl.loop(0, gpw, step=2)
        def _(j):
            d1 = pltpu.async_copy(table_hbm.at[idx_v.at[j + 1]], buf1, sem1)
            pltpu.make_async_copy(table_hbm.at[idx_v.at[j]], buf0, sem0).wait()
            accumulate(buf0, j)

            @pl.when(j + 2 < gpw)
            def _():
                pltpu.async_copy(table_hbm.at[idx_v.at[j + 2]], buf0, sem0)

            d1.wait()
            accumulate(buf1, j + 1)

        pltpu.sync_copy(out_v, out_hbm.at[pl.ds(wid * bpw, bpw)])

    return sc_pool


def _mm_body(p_ref, w_ref, b_ref, o_ref):
    o_ref[...] = (
        lax.dot_general(
            p_ref[...],
            w_ref[...],
            (((1,), (1,)), ((), ())),
            preferred_element_type=jnp.float32,
        )
        + b_ref[...]
    )


def _make_tc_matmul(B, EMB, OUT, bm, bn):
    grid = (B // bm, pl.cdiv(OUT, bn))
    return pl.pallas_call(
        _mm_body,
        grid=grid,
        in_specs=[
            pl.BlockSpec((bm, EMB), lambda i, j: (i, 0)),
            pl.BlockSpec((bn, EMB), lambda i, j: (j, 0)),
            pl.BlockSpec((1, bn), lambda i, j: (0, j)),
        ],
        out_specs=pl.BlockSpec((bm, bn), lambda i, j: (i, j)),
        out_shape=jax.ShapeDtypeStruct((B, OUT), jnp.float32),
        compiler_params=pltpu.CompilerParams(
            dimension_semantics=("parallel", "parallel"),
        ),
    )


def kernel(text_batch, table, W, b):
    B, L = text_batch.shape
    EMB = table.shape[1]
    OUT = W.shape[0]
    group = 2
    text2 = text_batch.astype(jnp.int32).reshape(B // group, group * L)
    pooled = _make_sc_pool(B, L, EMB, group)(text2, table)
    logits = _make_tc_matmul(B, EMB, OUT, 2048, 2048)(pooled, W, b.reshape(1, OUT))
    return logits


# speedup vs baseline: 1.0995x; 1.0995x over previous
import functools

import jax
import jax.numpy as jnp
from jax import lax
from jax.experimental import pallas as pl
from jax.experimental.pallas import tpu as pltpu
from jax.experimental.pallas import tpu_sc as plsc

_NUM_CORES = 2
_NUM_SUBCORES = 16
_NW = _NUM_CORES * _NUM_SUBCORES
_LANES = 16


def _make_sc_pool(B, Bc, L, EMB, base_row):
    assert Bc % (_NW * 2) == 0 and EMB % _LANES == 0
    bpw = Bc // _NW
    inv_l = 1.0 / float(L)
    mesh = plsc.VectorSubcoreMesh(core_axis_name="c", subcore_axis_name="s")

    @functools.partial(
        pl.kernel,
        out_type=jax.ShapeDtypeStruct((Bc, EMB), jnp.float32),
        mesh=mesh,
        scratch_types=[
            pltpu.VMEM((bpw, L), jnp.int32),
            pltpu.VMEM((bpw, EMB), jnp.float32),
            pltpu.VMEM((L, EMB), jnp.float32),
            pltpu.VMEM((L, EMB), jnp.float32),
            pltpu.SemaphoreType.DMA,
            pltpu.SemaphoreType.DMA,
        ],
    )
    def sc_pool(text_hbm, table_hbm, out_hbm, idx_v, out_v, buf0, buf1, sem0, sem1):
        wid = lax.axis_index("c") * _NUM_SUBCORES + lax.axis_index("s")
        pltpu.sync_copy(text_hbm.at[pl.ds(base_row + wid * bpw, bpw)], idx_v)

        def accumulate(buf, row):
            accs = [buf[0, pl.ds(cb * _LANES, _LANES)] for cb in range(EMB // _LANES)]
            for r in range(1, L):
                for cb in range(EMB // _LANES):
                    accs[cb] = accs[cb] + buf[r, pl.ds(cb * _LANES, _LANES)]
            for cb in range(EMB // _LANES):
                out_v[row, pl.ds(cb * _LANES, _LANES)] = accs[cb] * inv_l

        pltpu.async_copy(table_hbm.at[idx_v.at[0]], buf0, sem0)

        @pl.loop(0, bpw, step=2)
        def _(j):
            d1 = pltpu.async_copy(table_hbm.at[idx_v.at[j + 1]], buf1, sem1)
            pltpu.make_async_copy(table_hbm.at[idx_v.at[j]], buf0, sem0).wait()
            accumulate(buf0, j)

            @pl.when(j + 2 < bpw)
            def _():
                pltpu.async_copy(table_hbm.at[idx_v.at[j + 2]], buf0, sem0)

            d1.wait()
            accumulate(buf1, j + 1)

        pltpu.sync_copy(out_v, out_hbm.at[pl.ds(wid * bpw, bpw)])

    return sc_pool


def _mm_body(p_ref, w_ref, b_ref, o_ref):
    o_ref[...] = (
        lax.dot_general(
            p_ref[...],
            w_ref[...],
            (((1,), (1,)), ((), ())),
            preferred_element_type=jnp.float32,
        )
        + b_ref[...]
    )


def _mm_body_acc(p_ref, w_ref, b_ref, prev_ref, o_ref):
    del prev_ref
    _mm_body(p_ref, w_ref, b_ref, o_ref)


def _make_tc_matmul_chunk(B, Bc, EMB, OUT, bm, bn, chunk, first):
    grid = (Bc // bm, pl.cdiv(OUT, bn))
    row0 = chunk * (Bc // bm)
    in_specs = [
        pl.BlockSpec((bm, EMB), lambda i, j: (i, 0)),
        pl.BlockSpec((bn, EMB), lambda i, j: (j, 0)),
        pl.BlockSpec((1, bn), lambda i, j: (0, j)),
    ]
    if not first:
        in_specs.append(pl.BlockSpec((8, 128), lambda i, j: (0, 0)))
    return pl.pallas_call(
        _mm_body if first else _mm_body_acc,
        grid=grid,
        in_specs=in_specs,
        out_specs=pl.BlockSpec((bm, bn), lambda i, j: (row0 + i, j)),
        out_shape=jax.ShapeDtypeStruct((B, OUT), jnp.float32),
        input_output_aliases={} if first else {3: 0},
        compiler_params=pltpu.CompilerParams(
            dimension_semantics=("arbitrary", "arbitrary"),
        ),
    )


def kernel(text_batch, table, W, b):
    B, L = text_batch.shape
    EMB = table.shape[1]
    OUT = W.shape[0]
    nc = 4
    Bc = B // nc
    bm, bn = 1024, 2048
    text32 = text_batch.astype(jnp.int32)
    b2 = b.reshape(1, OUT)
    out = None
    for c in range(nc):
        pooled = _make_sc_pool(B, Bc, L, EMB, c * Bc)(text32, table)
        mm = _make_tc_matmul_chunk(B, Bc, EMB, OUT, bm, bn, c, first=(c == 0))
        if c == 0:
            out = mm(pooled, W, b2)
        else:
            out = mm(pooled, W, b2, out)
    return out


# speedup vs baseline: 1.2754x; 1.1600x over previous
import functools

import jax
import jax.numpy as jnp
from jax import lax
from jax.experimental import pallas as pl
from jax.experimental.pallas import tpu as pltpu
from jax.experimental.pallas import tpu_sc as plsc

_NUM_CORES = 2
_NUM_SUBCORES = 16
_NW = _NUM_CORES * _NUM_SUBCORES
_LANES = 16


def _make_sc_pool(B, L, EMB, group):
    assert B % (_NW * 2 * group) == 0 and EMB % _LANES == 0
    assert group * L <= 128
    bpw = B // _NW
    gpw = bpw // group
    gl = group * L
    inv_l = 1.0 / float(L)
    nvec = EMB // _LANES
    mesh = plsc.VectorSubcoreMesh(core_axis_name="c", subcore_axis_name="s")

    @functools.partial(
        pl.kernel,
        out_type=jax.ShapeDtypeStruct((B, EMB), jnp.float32),
        mesh=mesh,
        scratch_types=[
            pltpu.VMEM((gpw, gl), jnp.int32),
            pltpu.VMEM((bpw, EMB), jnp.float32),
            pltpu.VMEM((gl, EMB), jnp.float32),
            pltpu.VMEM((gl, EMB), jnp.float32),
            pltpu.SemaphoreType.DMA,
            pltpu.SemaphoreType.DMA,
        ],
    )
    def sc_pool(text_hbm, table_hbm, out_hbm, idx_v, out_v, buf0, buf1, sem0, sem1):
        wid = lax.axis_index("c") * _NUM_SUBCORES + lax.axis_index("s")
        pltpu.sync_copy(text_hbm.at[pl.ds(wid * gpw, gpw)], idx_v)

        def accumulate(buf, g):
            for e in range(group):
                init = tuple(
                    buf[e * L, pl.ds(cb * _LANES, _LANES)] for cb in range(nvec)
                )

                def body(r, accs):
                    return tuple(
                        accs[cb] + buf[e * L + r, pl.ds(cb * _LANES, _LANES)]
                        for cb in range(nvec)
                    )

                accs = lax.fori_loop(1, L, body, init, unroll=2)
                for cb in range(nvec):
                    out_v[g * group + e, pl.ds(cb * _LANES, _LANES)] = (
                        accs[cb] * inv_l
                    )

        pltpu.async_copy(table_hbm.at[idx_v.at[0]], buf0, sem0)

        @pl.loop(0, gpw, step=2)
        def _(j):
            d1 = pltpu.async_copy(table_hbm.at[idx_v.at[j + 1]], buf1, sem1)
            pltpu.make_async_copy(table_hbm.at[idx_v.at[j]], buf0, sem0).wait()
            accumulate(buf0, j)

            @pl.when(j + 2 < gpw)
            def _():
                pltpu.async_copy(table_hbm.at[idx_v.at[j + 2]], buf0, sem0)

            d1.wait()
            accumulate(buf1, j + 1)

        pltpu.sync_copy(out_v, out_hbm.at[pl.ds(wid * bpw, bpw)])

    return sc_pool


def _mm_body(p_ref, w_ref, b_ref, o_ref):
    o_ref[...] = (
        lax.dot_general(
            p_ref[...],
            w_ref[...],
            (((1,), (1,)), ((), ())),
            preferred_element_type=jnp.float32,
        )
        + b_ref[...]
    )


def _make_tc_matmul(B, EMB, OUT, bm, bn):
    grid = (B // bm, pl.cdiv(OUT, bn))
    return pl.pallas_call(
        _mm_body,
        grid=grid,
        in_specs=[
            pl.BlockSpec((bm, EMB), lambda i, j: (i, 0)),
            pl.BlockSpec((bn, EMB), lambda i, j: (j, 0)),
            pl.BlockSpec((1, bn), lambda i, j: (0, j)),
        ],
        out_specs=pl.BlockSpec((bm, bn), lambda i, j: (i, j)),
        out_shape=jax.ShapeDtypeStruct((B, OUT), jnp.float32),
        compiler_params=pltpu.CompilerParams(
            dimension_semantics=("parallel", "parallel"),
        ),
    )


def kernel(text_batch, table, W, b):
    B, L = text_batch.shape
    EMB = table.shape[1]
    OUT = W.shape[0]
    group = 2
    text2 = text_batch.astype(jnp.int32).reshape(B // group, group * L)
    pooled = _make_sc_pool(B, L, EMB, group)(text2, table)
    logits = _make_tc_matmul(B, EMB, OUT, 2048, 2048)(pooled, W, b.reshape(1, OUT))
    return logits
